# baseline (speedup 1.0000x reference)
import jax
import jax.numpy as jnp
from jax.experimental import pallas as pl
from jax.experimental.pallas import tpu as pltpu

_N = 2048
_F = 256
_NH = 8
_HD = 32
_C = 32
_R = 1024
_NB = _N // _R
_NEG = -1e9
_LOG2E = 1.4426950408889634


def _gat_kernel(x_ref, w1_ref, a1l_ref, a1r_ref, adj_ref, w2_ref, a2l_ref,
                a2r_ref, out_ref, gh_scr, el_scr, ert_scr, g2a_scr, aux_scr,
                auxt_scr):
    f32 = jnp.float32
    bf16 = jnp.bfloat16
    k = pl.program_id(0)

    @pl.when(k == 0)
    def _prologue():
        x = x_ref[...]
        g = jnp.dot(x, w1_ref[...], preferred_element_type=f32)
        a1l = a1l_ref[...] * _LOG2E
        a1r = a1r_ref[...] * _LOG2E
        r = jax.lax.broadcasted_iota(jnp.int32, (_F, _HD), 0)
        c = jax.lax.broadcasted_iota(jnp.int32, (_F, _HD), 1)
        ones = jnp.ones((_N, 1), bf16)
        els, ers = [], []
        for h in range(_NH):
            sel = (r == c + h * _HD).astype(f32)
            gh = jnp.dot(g, sel, preferred_element_type=f32)
            gh_scr[h] = jnp.concatenate([gh.astype(bf16), ones], axis=1)
            els.append(jnp.dot(gh, a1l, preferred_element_type=f32))
            ers.append(jnp.dot(gh, a1r, preferred_element_type=f32))
        el_scr[...] = jnp.concatenate(els, axis=1)
        ert_scr[...] = jnp.transpose(jnp.concatenate(ers, axis=1))

    mask = adj_ref[...]

    @pl.when(k < _NB)
    def _layer1():
        el = el_scr[pl.ds(k * _R, _R), :]
        ert = ert_scr[...]
        acc = jnp.zeros((_R, _C), f32)
        for h in range(_NH):
            s = el[:, h:h + 1] + ert[h:h + 1, :]
            s = jnp.maximum(s, 0.2 * s)
            s = jnp.where(mask, s, _NEG)
            m = jnp.max(s, axis=1, keepdims=True)
            p = jnp.exp2(s - m).astype(bf16)
            og = jnp.dot(p, gh_scr[h], preferred_element_type=f32)
            o = og[:, :_HD] / og[:, _HD:_HD + 1]
            o = jnp.where(o > 0, o, jnp.exp(o) - 1.0)
            w2h = w2_ref[pl.ds(h * _HD, _HD), :]
            acc = acc + jnp.dot(o, w2h, preferred_element_type=f32)
        g2a_scr[pl.ds(k * _R, _R), :] = jnp.concatenate(
            [acc.astype(bf16), jnp.ones((_R, 1), bf16)], axis=1)
        el2 = jnp.dot(acc, a2l_ref[...], preferred_element_type=f32) * _LOG2E
        er2 = jnp.dot(acc, a2r_ref[...], preferred_element_type=f32) * _LOG2E
        aux_scr[pl.ds(k * _R, _R), :] = jnp.concatenate([el2, er2], axis=1)

    @pl.when(k == _NB)
    def _transpose_aux():
        auxt_scr[...] = jnp.transpose(aux_scr[...])

    @pl.when(k >= _NB)
    def _layer2():
        el2 = aux_scr[pl.ds((k - _NB) * _R, _R), 0:1]
        s = el2 + auxt_scr[1:2, :]
        s = jnp.maximum(s, 0.2 * s)
        s = jnp.where(mask, s, _NEG)
        m = jnp.max(s, axis=1, keepdims=True)
        p = jnp.exp2(s - m).astype(bf16)
        og = jnp.dot(p, g2a_scr[...], preferred_element_type=f32)
        out_ref[...] = og[:, :_C] / og[:, _C:_C + 1]


def kernel(x, adj_mat, W1, a1_l, a1_r, W2, a2_l, a2_r):
    f32 = jnp.float32
    adj = adj_mat.reshape(_N, _N)

    out = pl.pallas_call(
        _gat_kernel,
        grid=(2 * _NB,),
        in_specs=[
            pl.BlockSpec((_N, _F), lambda k: (0, 0)),
            pl.BlockSpec((_F, _F), lambda k: (0, 0)),
            pl.BlockSpec((_HD, 1), lambda k: (0, 0)),
            pl.BlockSpec((_HD, 1), lambda k: (0, 0)),
            pl.BlockSpec((_R, _N), lambda k: (jax.lax.rem(k, _NB), 0)),
            pl.BlockSpec((_F, _C), lambda k: (0, 0)),
            pl.BlockSpec((_C, 1), lambda k: (0, 0)),
            pl.BlockSpec((_C, 1), lambda k: (0, 0)),
        ],
        out_specs=pl.BlockSpec(
            (_R, _C), lambda k: (jnp.maximum(k - _NB, 0), 0)),
        out_shape=jax.ShapeDtypeStruct((_N, _C), f32),
        scratch_shapes=[
            pltpu.VMEM((_NH, _N, _HD + 1), jnp.bfloat16),
            pltpu.VMEM((_N, _NH), f32),
            pltpu.VMEM((_NH, _N), f32),
            pltpu.VMEM((_N, _C + 1), jnp.bfloat16),
            pltpu.VMEM((_N, 2), f32),
            pltpu.VMEM((2, _N), f32),
        ],
    )(x, W1, a1_l.reshape(_HD, 1), a1_r.reshape(_HD, 1), adj, W2,
      a2_l.reshape(_C, 1), a2_r.reshape(_C, 1))

    return out


# speedup vs baseline: 1.7290x; 1.7290x over previous
import jax
import jax.numpy as jnp
from jax.experimental import pallas as pl
from jax.experimental.pallas import tpu as pltpu

_N = 2048
_F = 256
_NH = 8
_HD = 32
_C = 32
_R = 512
_NB = _N // _R
_NEG = -1e9
_LOG2E = 1.4426950408889634


def _gat_kernel(x_ref, w1_ref, a1l_ref, a1r_ref, adj_ref, w2_ref, a2l_ref,
                a2r_ref, out_ref, gh_scr, el_scr, ert_scr, g2a_scr, aux_scr,
                auxt_scr, ermax_scr):
    f32 = jnp.float32
    bf16 = jnp.bfloat16
    k = pl.program_id(0)

    @pl.when(k == 0)
    def _prologue():
        x = x_ref[...]
        g = jnp.dot(x, w1_ref[...], preferred_element_type=f32)
        a1lr = jnp.concatenate([a1l_ref[...], a1r_ref[...]],
                               axis=1) * _LOG2E
        r = jax.lax.broadcasted_iota(jnp.int32, (_F, _HD), 0)
        c = jax.lax.broadcasted_iota(jnp.int32, (_F, _HD), 1)
        ones = jnp.ones((_N, 1), bf16)
        els, ers = [], []
        for h in range(_NH):
            sel = (r == c + h * _HD).astype(f32)
            gh = jnp.dot(g, sel, preferred_element_type=f32)
            gh_scr[h] = jnp.concatenate([gh.astype(bf16), ones], axis=1)
            elr = jnp.dot(gh, a1lr, preferred_element_type=f32)
            els.append(elr[:, 0:1])
            ers.append(elr[:, 1:2])
        el_scr[...] = jnp.concatenate(els, axis=1)
        ert = jnp.transpose(jnp.concatenate(ers, axis=1))
        ert_scr[...] = ert
        ermax_scr[...] = jnp.max(ert, axis=1, keepdims=True)

    mask = adj_ref[...]

    @pl.when(k < _NB)
    def _layer1():
        el = el_scr[pl.ds(k * _R, _R), :]
        ert = ert_scr[...]
        acc = jnp.zeros((_R, _C), f32)
        for h in range(_NH):
            elh = el[:, h:h + 1]
            s = elh + ert[h:h + 1, :]
            s = jnp.maximum(s, 0.2 * s)
            v = elh + ermax_scr[h:h + 1, :]
            u = jnp.maximum(v, 0.2 * v)
            s = jnp.where(mask, s - u, _NEG)
            p = jnp.exp2(s).astype(bf16)
            og = jnp.dot(p, gh_scr[h], preferred_element_type=f32)
            o = og[:, :_HD] / jnp.maximum(og[:, _HD:_HD + 1], 1e-30)
            o = jnp.where(o > 0, o, jnp.exp(o) - 1.0)
            w2h = w2_ref[pl.ds(h * _HD, _HD), :]
            acc = acc + jnp.dot(o, w2h, preferred_element_type=f32)
        g2a_scr[pl.ds(k * _R, _R), :] = jnp.concatenate(
            [acc.astype(bf16), jnp.ones((_R, 1), bf16)], axis=1)
        a2lr = jnp.concatenate([a2l_ref[...], a2r_ref[...]], axis=1)
        aux_scr[pl.ds(k * _R, _R), :] = jnp.dot(
            acc, a2lr, preferred_element_type=f32) * _LOG2E

    @pl.when(k == _NB)
    def _transpose_aux():
        auxt_scr[...] = jnp.transpose(aux_scr[...])

    @pl.when(k >= _NB)
    def _layer2():
        el2 = aux_scr[pl.ds((k - _NB) * _R, _R), 0:1]
        er2t = auxt_scr[1:2, :]
        um = jnp.max(er2t, axis=1, keepdims=True)
        v = el2 + um
        u = jnp.maximum(v, 0.2 * v)
        s = el2 + er2t
        s = jnp.maximum(s, 0.2 * s)
        s = jnp.where(mask, s - u, _NEG)
        p = jnp.exp2(s).astype(bf16)
        og = jnp.dot(p, g2a_scr[...], preferred_element_type=f32)
        out_ref[...] = og[:, :_C] / jnp.maximum(og[:, _C:_C + 1], 1e-30)


def kernel(x, adj_mat, W1, a1_l, a1_r, W2, a2_l, a2_r):
    f32 = jnp.float32
    adj = adj_mat.reshape(_N, _N)

    out = pl.pallas_call(
        _gat_kernel,
        grid=(2 * _NB,),
        in_specs=[
            pl.BlockSpec((_N, _F), lambda k: (0, 0)),
            pl.BlockSpec((_F, _F), lambda k: (0, 0)),
            pl.BlockSpec((_HD, 1), lambda k: (0, 0)),
            pl.BlockSpec((_HD, 1), lambda k: (0, 0)),
            pl.BlockSpec((_R, _N), lambda k: (jax.lax.rem(k, _NB), 0)),
            pl.BlockSpec((_F, _C), lambda k: (0, 0)),
            pl.BlockSpec((_C, 1), lambda k: (0, 0)),
            pl.BlockSpec((_C, 1), lambda k: (0, 0)),
        ],
        out_specs=pl.BlockSpec(
            (_R, _C), lambda k: (jnp.maximum(k - _NB, 0), 0)),
        out_shape=jax.ShapeDtypeStruct((_N, _C), f32),
        scratch_shapes=[
            pltpu.VMEM((_NH, _N, _HD + 1), jnp.bfloat16),
            pltpu.VMEM((_N, _NH), f32),
            pltpu.VMEM((_NH, _N), f32),
            pltpu.VMEM((_N, _C + 1), jnp.bfloat16),
            pltpu.VMEM((_N, 2), f32),
            pltpu.VMEM((2, _N), f32),
            pltpu.VMEM((_NH, 1), f32),
        ],
    )(x, W1, a1_l.reshape(_HD, 1), a1_r.reshape(_HD, 1), adj, W2,
      a2_l.reshape(_C, 1), a2_r.reshape(_C, 1))

    return out
